# baseline (speedup 1.0000x reference)
import functools

import jax
import jax.numpy as jnp
from jax import lax
from jax.experimental import pallas as pl
from jax.experimental.pallas import tpu as pltpu
from jax.experimental.pallas import tpu_sc as plsc

F32 = jnp.float32
H = 128
BR = 400

_MESH = plsc.VectorSubcoreMesh(core_axis_name="c", subcore_axis_name="s")
NSC = 2
NTEC = 16
CH = 128


def _enc_body(x_ref, w_ref, b_ref, o_ref):
    o_ref[...] = (
        jnp.dot(x_ref[...], w_ref[...], preferred_element_type=F32) + b_ref[...]
    )


def _encode(x, W, b):
    N, Fin = x.shape
    return pl.pallas_call(
        _enc_body,
        grid=(N // BR,),
        in_specs=[
            pl.BlockSpec((BR, Fin), lambda i: (i, 0)),
            pl.BlockSpec((Fin, H), lambda i: (0, 0)),
            pl.BlockSpec((1, H), lambda i: (0, 0)),
        ],
        out_specs=pl.BlockSpec((BR, H), lambda i: (i, 0)),
        out_shape=jax.ShapeDtypeStruct((N, H), F32),
    )(x, W, b.reshape(1, H))


def _comb1_body(sa, sb, ca, cb, h, wl, wr, b, o, *, relu):
    s = sa[0] + sb[0]
    c = jnp.sum(ca[0] + cb[0], axis=1, keepdims=True)
    mean = s * (128.0 / jnp.maximum(c, 128.0))
    r = (
        jnp.dot(mean, wl[...], preferred_element_type=F32)
        + jnp.dot(h[...], wr[...], preferred_element_type=F32)
        + b[...]
    )
    o[...] = jnp.maximum(r, 0.0) if relu else r


def _combine1(s2, c2, h, wl, wr, b, relu):
    N = h.shape[0]
    return pl.pallas_call(
        functools.partial(_comb1_body, relu=relu),
        grid=(N // BR,),
        in_specs=[
            pl.BlockSpec((1, BR, H), lambda i: (0, i, 0)),
            pl.BlockSpec((1, BR, H), lambda i: (1, i, 0)),
            pl.BlockSpec((1, BR, H), lambda i: (0, i, 0)),
            pl.BlockSpec((1, BR, H), lambda i: (1, i, 0)),
            pl.BlockSpec((BR, H), lambda i: (i, 0)),
            pl.BlockSpec((H, H), lambda i: (0, 0)),
            pl.BlockSpec((H, H), lambda i: (0, 0)),
            pl.BlockSpec((1, H), lambda i: (0, 0)),
        ],
        out_specs=pl.BlockSpec((BR, H), lambda i: (i, 0)),
        out_shape=jax.ShapeDtypeStruct((N, H), F32),
    )(s2, s2, c2, c2, h, wl, wr, b.reshape(1, H))


def _comb2_body(sa, sb, ca, cb, s2a, s2b, c2a, c2b, h, wl, wl2, wr, b, o,
                *, relu, nb2):
    s = sa[0] + sb[0]
    c = jnp.sum(ca[0] + cb[0], axis=1, keepdims=True)
    mean = s * (128.0 / jnp.maximum(c, 128.0))
    s2 = s2a[0] + s2b[0]
    c2 = jnp.sum(c2a[0] + c2b[0], axis=1, keepdims=True)
    mean2 = s2 * (128.0 / jnp.maximum(c2, 128.0))
    valid = (pl.program_id(0) < nb2).astype(F32)
    r = (
        jnp.dot(mean, wl[...], preferred_element_type=F32)
        + jnp.dot(mean2 * valid, wl2[...], preferred_element_type=F32)
        + jnp.dot(h[...], wr[...], preferred_element_type=F32)
        + b[...]
    )
    o[...] = jnp.maximum(r, 0.0) if relu else r


def _combine2(s2, c2, sv2, cv2, h, wl, wl2, wr, b, relu):
    N = h.shape[0]
    nb2 = sv2.shape[1] // BR

    def clamp(part):
        return lambda i: (part, jnp.minimum(i, nb2 - 1), 0)

    return pl.pallas_call(
        functools.partial(_comb2_body, relu=relu, nb2=nb2),
        grid=(N // BR,),
        in_specs=[
            pl.BlockSpec((1, BR, H), lambda i: (0, i, 0)),
            pl.BlockSpec((1, BR, H), lambda i: (1, i, 0)),
            pl.BlockSpec((1, BR, H), lambda i: (0, i, 0)),
            pl.BlockSpec((1, BR, H), lambda i: (1, i, 0)),
            pl.BlockSpec((1, BR, H), clamp(0)),
            pl.BlockSpec((1, BR, H), clamp(1)),
            pl.BlockSpec((1, BR, H), clamp(0)),
            pl.BlockSpec((1, BR, H), clamp(1)),
            pl.BlockSpec((BR, H), lambda i: (i, 0)),
            pl.BlockSpec((H, H), lambda i: (0, 0)),
            pl.BlockSpec((H, H), lambda i: (0, 0)),
            pl.BlockSpec((H, H), lambda i: (0, 0)),
            pl.BlockSpec((1, H), lambda i: (0, 0)),
        ],
        out_specs=pl.BlockSpec((BR, H), lambda i: (i, 0)),
        out_shape=jax.ShapeDtypeStruct((N, H), F32),
    )(s2, s2, c2, c2, sv2, sv2, cv2, cv2, h, wl, wl2, wr, b.reshape(1, H))


def _segsum(y, src, dst, E_pad, NP, D_TILE):
    E2 = E_pad // NSC
    NCH = E2 // (NTEC * CH)
    EPT = NCH * CH
    ACC = ((D_TILE + 16) + CH - 1) // CH * CH
    ZR = ACC // NTEC
    FR = D_TILE // NTEC
    ND = NP * D_TILE

    @functools.partial(
        pl.kernel,
        mesh=_MESH,
        out_type=jax.ShapeDtypeStruct((NSC, ND, H), F32),
        scratch_types=[
            pltpu.VMEM((CH, H), F32),
            pltpu.VMEM((CH,), jnp.int32),
            pltpu.VMEM((1, CH), jnp.int32),
            pltpu.VMEM((1, CH), jnp.int32),
            pltpu.VMEM_SHARED((ACC, H), F32),
            pltpu.SemaphoreType.DMA,
        ],
    )
    def seg(y_hbm, src_hbm, dst_hbm, out_hbm, payload, srcb, dstb,
            offb, acc, sem):
        c = lax.axis_index("c")
        s = lax.axis_index("s")
        zv = jnp.zeros((16,), F32)
        e_tec = c * E2 + s * EPT

        for p in range(NP):
            lo = p * D_TILE

            def zrow(i, _):
                for j in range(H // 16):
                    payload[i, pl.ds(j * 16, 16)] = zv
                return 0

            lax.fori_loop(0, CH, zrow, 0)
            zbase = s * ZR
            for k in range(ZR // CH):
                pltpu.sync_copy(payload, acc.at[pl.ds(zbase + k * CH, CH)])
            if ZR % CH:
                pltpu.sync_copy(
                    payload.at[pl.ds(0, ZR % CH)],
                    acc.at[pl.ds(zbase + (ZR // CH) * CH, ZR % CH)],
                )
            plsc.subcore_barrier()

            def chunk(g, _, lo=lo):
                e_off = pl.multiple_of(e_tec + g * CH, CH)
                pltpu.sync_copy(dst_hbm.at[pl.ds(e_off, CH)], dstb.at[0])
                pltpu.sync_copy(src_hbm.at[pl.ds(e_off, CH)], srcb)
                for j in range(CH // 16):
                    d = dstb[0, pl.ds(j * 16, 16)]
                    inr = (d >= lo) & (d < lo + D_TILE)
                    offb[0, pl.ds(j * 16, 16)] = jnp.where(inr, d - lo, D_TILE)
                pltpu.async_copy(y_hbm.at[srcb], payload, sem).wait()
                pltpu.sync_copy(payload, acc.at[offb.at[0]], add=True)
                return 0

            lax.fori_loop(0, NCH, chunk, 0)
            plsc.subcore_barrier()
            pltpu.sync_copy(
                acc.at[pl.ds(s * FR, FR)],
                out_hbm.at[c, pl.ds(lo + s * FR, FR)],
            )
            plsc.subcore_barrier()

    return seg(y, src, dst)


def _pad_edges(ei, pad_dst):
    E = ei.shape[1]
    unit = NSC * NTEC * CH
    E_pad = (E + unit - 1) // unit * unit
    src = jnp.concatenate([ei[0], jnp.zeros((E_pad - E,), jnp.int32)])
    dst = jnp.concatenate(
        [ei[1], jnp.full((E_pad - E,), pad_dst, jnp.int32)]
    )
    return src, dst, E_pad


def kernel(x_user, x_travel, x_visit_area, ei_ut, ei_tu, ei_tv, ei_vt,
           We_u, be_u, We_t, be_t, We_v, be_v, W1l, W1r, b1, W2l, W2r, b2):
    NT_TILE = 12544
    NP_T = 4
    NV_TILE = 10112
    PAD_T, PAD_V = NP_T * NT_TILE, NV_TILE

    src_ut, dst_ut, Eut = _pad_edges(ei_ut, PAD_T)
    src_tu, dst_tu, Etu = _pad_edges(ei_tu, PAD_T)
    src_tv, dst_tv, Etv = _pad_edges(ei_tv, PAD_V)
    src_vt, dst_vt, Evt = _pad_edges(ei_vt, PAD_V)

    ones_tab = jnp.ones((8, H), F32)
    zsrc_t = jnp.zeros((Eut,), jnp.int32)
    zsrc_v = jnp.zeros((Etv,), jnp.int32)
    c_ut = _segsum(ones_tab, zsrc_t, dst_ut, Eut, NP_T, NT_TILE)
    c_tu = _segsum(ones_tab, zsrc_t, dst_tu, Etu, NP_T, NT_TILE)
    c_tv = _segsum(ones_tab, zsrc_v, dst_tv, Etv, 1, NV_TILE)
    c_vt = _segsum(ones_tab, zsrc_v, dst_vt, Evt, 1, NV_TILE)

    h_u = _encode(x_user, We_u, be_u)
    h_t = _encode(x_travel, We_t, be_t)
    h_v = _encode(x_visit_area, We_v, be_v)

    def layer(hu, ht, hv, Wl, Wr, b, relu):
        s_ut = _segsum(hu, src_ut, dst_ut, Eut, NP_T, NT_TILE)
        s_vt = _segsum(hv, src_vt, dst_vt, Evt, 1, NV_TILE)
        s_tu = _segsum(ht, src_tu, dst_tu, Etu, NP_T, NT_TILE)
        s_tv = _segsum(ht, src_tv, dst_tv, Etv, 1, NV_TILE)
        t = _combine2(s_ut, c_ut, s_vt, c_vt, ht,
                      Wl[0], Wl[3], Wr[0] + Wr[3], b[0] + b[3], relu)
        u = _combine1(s_tu, c_tu, hu, Wl[1], Wr[1], b[1], relu)
        v = _combine1(s_tv, c_tv, hv, Wl[2], Wr[2], b[2], relu)
        return u, t, v

    u1, t1, v1 = layer(h_u, h_t, h_v, W1l, W1r, b1, True)
    u2, t2, v2 = layer(u1, t1, v1, W2l, W2r, b2, False)
    return (u2, t2, v2)


# speedup vs baseline: 17.3675x; 17.3675x over previous
import functools

import jax
import jax.numpy as jnp
from jax import lax
from jax.experimental import pallas as pl
from jax.experimental.pallas import tpu as pltpu
from jax.experimental.pallas import tpu_sc as plsc

F32 = jnp.float32
H = 128
BR = 400

_MESH = plsc.VectorSubcoreMesh(core_axis_name="c", subcore_axis_name="s")
NSC = 2
NTEC = 16
CH = 128


def _enc_body(x_ref, w_ref, b_ref, o_ref):
    o_ref[...] = (
        jnp.dot(x_ref[...], w_ref[...], preferred_element_type=F32) + b_ref[...]
    )


def _encode(x, W, b):
    N, Fin = x.shape
    return pl.pallas_call(
        _enc_body,
        grid=(N // BR,),
        in_specs=[
            pl.BlockSpec((BR, Fin), lambda i: (i, 0)),
            pl.BlockSpec((Fin, H), lambda i: (0, 0)),
            pl.BlockSpec((1, H), lambda i: (0, 0)),
        ],
        out_specs=pl.BlockSpec((BR, H), lambda i: (i, 0)),
        out_shape=jax.ShapeDtypeStruct((N, H), F32),
    )(x, W, b.reshape(1, H))


def _comb1_body(sa, sb, ca, cb, h, wl, wr, b, o, *, relu):
    s = sa[0] + sb[0]
    c = jnp.sum(ca[0] + cb[0], axis=1, keepdims=True)
    mean = s * (128.0 / jnp.maximum(c, 128.0))
    r = (
        jnp.dot(mean, wl[...], preferred_element_type=F32)
        + jnp.dot(h[...], wr[...], preferred_element_type=F32)
        + b[...]
    )
    o[...] = jnp.maximum(r, 0.0) if relu else r


def _combine1(s2, c2, h, wl, wr, b, relu):
    N = h.shape[0]
    return pl.pallas_call(
        functools.partial(_comb1_body, relu=relu),
        grid=(N // BR,),
        in_specs=[
            pl.BlockSpec((1, BR, H), lambda i: (0, i, 0)),
            pl.BlockSpec((1, BR, H), lambda i: (1, i, 0)),
            pl.BlockSpec((1, BR, H), lambda i: (0, i, 0)),
            pl.BlockSpec((1, BR, H), lambda i: (1, i, 0)),
            pl.BlockSpec((BR, H), lambda i: (i, 0)),
            pl.BlockSpec((H, H), lambda i: (0, 0)),
            pl.BlockSpec((H, H), lambda i: (0, 0)),
            pl.BlockSpec((1, H), lambda i: (0, 0)),
        ],
        out_specs=pl.BlockSpec((BR, H), lambda i: (i, 0)),
        out_shape=jax.ShapeDtypeStruct((N, H), F32),
    )(s2, s2, c2, c2, h, wl, wr, b.reshape(1, H))


def _comb2_body(sa, sb, ca, cb, s2a, s2b, c2a, c2b, h, wl, wl2, wr, b, o,
                *, relu, nb2):
    s = sa[0] + sb[0]
    c = jnp.sum(ca[0] + cb[0], axis=1, keepdims=True)
    mean = s * (128.0 / jnp.maximum(c, 128.0))
    s2 = s2a[0] + s2b[0]
    c2 = jnp.sum(c2a[0] + c2b[0], axis=1, keepdims=True)
    mean2 = s2 * (128.0 / jnp.maximum(c2, 128.0))
    valid = (pl.program_id(0) < nb2).astype(F32)
    r = (
        jnp.dot(mean, wl[...], preferred_element_type=F32)
        + jnp.dot(mean2 * valid, wl2[...], preferred_element_type=F32)
        + jnp.dot(h[...], wr[...], preferred_element_type=F32)
        + b[...]
    )
    o[...] = jnp.maximum(r, 0.0) if relu else r


def _combine2(s2, c2, sv2, cv2, h, wl, wl2, wr, b, relu):
    N = h.shape[0]
    nb2 = sv2.shape[1] // BR

    def clamp(part):
        return lambda i: (part, jnp.minimum(i, nb2 - 1), 0)

    return pl.pallas_call(
        functools.partial(_comb2_body, relu=relu, nb2=nb2),
        grid=(N // BR,),
        in_specs=[
            pl.BlockSpec((1, BR, H), lambda i: (0, i, 0)),
            pl.BlockSpec((1, BR, H), lambda i: (1, i, 0)),
            pl.BlockSpec((1, BR, H), lambda i: (0, i, 0)),
            pl.BlockSpec((1, BR, H), lambda i: (1, i, 0)),
            pl.BlockSpec((1, BR, H), clamp(0)),
            pl.BlockSpec((1, BR, H), clamp(1)),
            pl.BlockSpec((1, BR, H), clamp(0)),
            pl.BlockSpec((1, BR, H), clamp(1)),
            pl.BlockSpec((BR, H), lambda i: (i, 0)),
            pl.BlockSpec((H, H), lambda i: (0, 0)),
            pl.BlockSpec((H, H), lambda i: (0, 0)),
            pl.BlockSpec((H, H), lambda i: (0, 0)),
            pl.BlockSpec((1, H), lambda i: (0, 0)),
        ],
        out_specs=pl.BlockSpec((BR, H), lambda i: (i, 0)),
        out_shape=jax.ShapeDtypeStruct((N, H), F32),
    )(s2, s2, c2, c2, sv2, sv2, cv2, cv2, h, wl, wl2, wr, b.reshape(1, H))


def _segsum(y, src, dst, E_pad, NP, D_TILE, gather=True):
    E2 = E_pad // NSC
    NCH = E2 // (NTEC * CH)
    EPT = NCH * CH
    ACC = ((D_TILE + 16) + CH - 1) // CH * CH
    ZR = ACC // NTEC
    FR = D_TILE // NTEC
    ND = NP * D_TILE

    scratch = [
        pltpu.VMEM((CH, H), F32),
        pltpu.VMEM((1, CH), jnp.int32),
        pltpu.VMEM((1, CH), jnp.int32),
        pltpu.VMEM_SHARED((ACC, H), F32),
    ]
    if gather:
        scratch += [
            pltpu.VMEM((CH,), jnp.int32),
            pltpu.SemaphoreType.DMA,
        ]

    @functools.partial(
        pl.kernel,
        mesh=_MESH,
        out_type=jax.ShapeDtypeStruct((NSC, ND, H), F32),
        scratch_types=scratch,
    )
    def seg(*refs):
        if gather:
            y_hbm, src_hbm, dst_hbm, out_hbm, payload, dstb, offb, acc, \
                srcb, sem = refs
        else:
            dst_hbm, out_hbm, payload, dstb, offb, acc = refs
        c = lax.axis_index("c")
        s = lax.axis_index("s")
        zv = jnp.zeros((16,), F32)
        ov = jnp.full((16,), 1.0, F32)
        e_tec = c * E2 + s * EPT

        def fill(val):
            def row(i, _):
                for j in range(H // 16):
                    payload[i, pl.ds(j * 16, 16)] = val
                return 0
            lax.fori_loop(0, CH, row, 0)

        for p in range(NP):
            lo = p * D_TILE

            fill(zv)
            zbase = s * ZR
            for k in range(ZR // CH):
                pltpu.sync_copy(payload, acc.at[pl.ds(zbase + k * CH, CH)])
            if ZR % CH:
                pltpu.sync_copy(
                    payload.at[pl.ds(0, ZR % CH)],
                    acc.at[pl.ds(zbase + (ZR // CH) * CH, ZR % CH)],
                )
            if not gather:
                fill(ov)
            plsc.subcore_barrier()

            def chunk(g, _, lo=lo):
                e_off = pl.multiple_of(e_tec + g * CH, CH)
                pltpu.sync_copy(dst_hbm.at[pl.ds(e_off, CH)], dstb.at[0])
                if gather:
                    pltpu.sync_copy(src_hbm.at[pl.ds(e_off, CH)], srcb)
                for j in range(CH // 16):
                    d = dstb[0, pl.ds(j * 16, 16)]
                    inr = (d >= lo) & (d < lo + D_TILE)
                    offb[0, pl.ds(j * 16, 16)] = jnp.where(inr, d - lo, D_TILE)
                if gather:
                    pltpu.async_copy(y_hbm.at[srcb], payload, sem).wait()
                pltpu.sync_copy(payload, acc.at[offb.at[0]], add=True)
                return 0

            lax.fori_loop(0, NCH, chunk, 0)
            plsc.subcore_barrier()
            pltpu.sync_copy(
                acc.at[pl.ds(s * FR, FR)],
                out_hbm.at[c, pl.ds(lo + s * FR, FR)],
            )
            plsc.subcore_barrier()

    if gather:
        return seg(y, src, dst)
    return seg(dst)


def _pad_edges(ei, pad_dst):
    E = ei.shape[1]
    unit = NSC * NTEC * CH
    E_pad = (E + unit - 1) // unit * unit
    src = jnp.concatenate([ei[0], jnp.zeros((E_pad - E,), jnp.int32)])
    dst = jnp.concatenate(
        [ei[1], jnp.full((E_pad - E,), pad_dst, jnp.int32)]
    )
    return src, dst, E_pad


def kernel(x_user, x_travel, x_visit_area, ei_ut, ei_tu, ei_tv, ei_vt,
           We_u, be_u, We_t, be_t, We_v, be_v, W1l, W1r, b1, W2l, W2r, b2):
    NT_TILE = 12544
    NP_T = 4
    NV_TILE = 10112
    PAD_T, PAD_V = NP_T * NT_TILE, NV_TILE

    src_ut, dst_ut, Eut = _pad_edges(ei_ut, PAD_T)
    src_tu, dst_tu, Etu = _pad_edges(ei_tu, PAD_T)
    src_tv, dst_tv, Etv = _pad_edges(ei_tv, PAD_V)
    src_vt, dst_vt, Evt = _pad_edges(ei_vt, PAD_V)

    c_ut = _segsum(None, None, dst_ut, Eut, NP_T, NT_TILE, gather=False)
    c_tu = _segsum(None, None, dst_tu, Etu, NP_T, NT_TILE, gather=False)
    c_tv = _segsum(None, None, dst_tv, Etv, 1, NV_TILE, gather=False)
    c_vt = _segsum(None, None, dst_vt, Evt, 1, NV_TILE, gather=False)

    h_u = _encode(x_user, We_u, be_u)
    h_t = _encode(x_travel, We_t, be_t)
    h_v = _encode(x_visit_area, We_v, be_v)

    def layer(hu, ht, hv, Wl, Wr, b, relu):
        s_ut = _segsum(hu, src_ut, dst_ut, Eut, NP_T, NT_TILE)
        s_vt = _segsum(hv, src_vt, dst_vt, Evt, 1, NV_TILE)
        s_tu = _segsum(ht, src_tu, dst_tu, Etu, NP_T, NT_TILE)
        s_tv = _segsum(ht, src_tv, dst_tv, Etv, 1, NV_TILE)
        t = _combine2(s_ut, c_ut, s_vt, c_vt, ht,
                      Wl[0], Wl[3], Wr[0] + Wr[3], b[0] + b[3], relu)
        u = _combine1(s_tu, c_tu, hu, Wl[1], Wr[1], b[1], relu)
        v = _combine1(s_tv, c_tv, hv, Wl[2], Wr[2], b[2], relu)
        return u, t, v

    u1, t1, v1 = layer(h_u, h_t, h_v, W1l, W1r, b1, True)
    u2, t2, v2 = layer(u1, t1, v1, W2l, W2r, b2, False)
    return (u2, t2, v2)
